# baseline (speedup 1.0000x reference)
import jax
import jax.numpy as jnp
from jax.experimental import pallas as pl
from jax.experimental.pallas import tpu as pltpu

_NUM_CLS = 100000
_SCALE = 32.0
_ALPHA = 0.2
_NEG = -1e30

_RBLK = 512
_CBLK = 2048


def _cos_loss_body(y_ref, x_ref, o_ref, m_ref, s_ref, t_ref):
    j = pl.program_id(1)
    ncol = pl.num_programs(1)

    @pl.when(j == 0)
    def _init():
        m_ref[...] = jnp.full_like(m_ref, _NEG)
        s_ref[...] = jnp.zeros_like(s_ref)
        t_ref[...] = jnp.zeros_like(t_ref)

    x = x_ref[...]
    cols = jax.lax.broadcasted_iota(jnp.int32, x.shape, 1) + j * _CBLK
    x = jnp.where(cols < _NUM_CLS, x, _NEG)

    m_old = m_ref[...]
    bm = jnp.max(x, axis=1, keepdims=True)
    m_new = jnp.maximum(m_old, bm)
    p = jnp.exp(_SCALE * (x - m_new))
    s_ref[...] = s_ref[...] * jnp.exp(_SCALE * (m_old - m_new)) + jnp.sum(
        p, axis=1, keepdims=True
    )
    m_ref[...] = m_new

    y = y_ref[...]
    t_ref[...] = t_ref[...] + jnp.sum(
        jnp.where(cols == y, x, 0.0), axis=1, keepdims=True
    )

    @pl.when(j == ncol - 1)
    def _finish():
        m = m_ref[...]
        s = s_ref[...]
        t = t_ref[...]
        tgt = _SCALE * (t - _ALPHA)
        s_adj = s - jnp.exp(_SCALE * (t - m)) + jnp.exp(tgt - _SCALE * m)
        o_ref[...] = _SCALE * m + jnp.log(s_adj) - tgt


def kernel(score, y):
    batch, num_cls = score.shape
    y2 = y.reshape(-1, 1).astype(jnp.int32)
    ncol = pl.cdiv(num_cls, _CBLK)
    out = pl.pallas_call(
        _cos_loss_body,
        grid=(batch // _RBLK, ncol),
        in_specs=[
            pl.BlockSpec((_RBLK, 1), lambda i, j: (i, 0)),
            pl.BlockSpec((_RBLK, _CBLK), lambda i, j: (i, j)),
        ],
        out_specs=pl.BlockSpec((_RBLK, 1), lambda i, j: (i, 0)),
        out_shape=jax.ShapeDtypeStruct((batch, 1), score.dtype),
        scratch_shapes=[
            pltpu.VMEM((_RBLK, 1), jnp.float32),
            pltpu.VMEM((_RBLK, 1), jnp.float32),
            pltpu.VMEM((_RBLK, 1), jnp.float32),
        ],
        compiler_params=pltpu.CompilerParams(
            dimension_semantics=("parallel", "arbitrary"),
        ),
    )(y2, score)
    return out[:, 0]


# speedup vs baseline: 1.0162x; 1.0162x over previous
import jax
import jax.numpy as jnp
from jax.experimental import pallas as pl
from jax.experimental.pallas import tpu as pltpu

_NUM_CLS = 100000
_SCALE = 32.0
_ALPHA = 0.2
_NEG = -1e30

_RBLK = 512
_CBLK = 2048


def _cos_loss_body(y_ref, x_ref, o_ref, m_ref, s_ref, t_ref):
    j = pl.program_id(1)
    ncol = pl.num_programs(1)

    @pl.when(j == 0)
    def _init():
        m_ref[...] = jnp.full_like(m_ref, _NEG)
        s_ref[...] = jnp.zeros_like(s_ref)
        t_ref[...] = jnp.zeros_like(t_ref)

    def _update(x, cols):
        m_old = m_ref[...]
        bm = jnp.max(x, axis=1, keepdims=True)
        m_new = jnp.maximum(m_old, bm)
        p = jnp.exp(_SCALE * (x - m_new))
        s_ref[...] = s_ref[...] * jnp.exp(_SCALE * (m_old - m_new)) + jnp.sum(
            p, axis=1, keepdims=True
        )
        m_ref[...] = m_new
        y = y_ref[...]
        t_ref[...] = t_ref[...] + jnp.sum(
            jnp.where(cols == y, x, 0.0), axis=1, keepdims=True
        )

    @pl.when(j < ncol - 1)
    def _steady():
        x = x_ref[...]
        cols = jax.lax.broadcasted_iota(jnp.int32, x.shape, 1) + j * _CBLK
        _update(x, cols)

    @pl.when(j == ncol - 1)
    def _tail():
        x = x_ref[...]
        cols = jax.lax.broadcasted_iota(jnp.int32, x.shape, 1) + j * _CBLK
        x = jnp.where(cols < _NUM_CLS, x, _NEG)
        _update(x, cols)

    @pl.when(j == ncol - 1)
    def _finish():
        m = m_ref[...]
        s = s_ref[...]
        t = t_ref[...]
        tgt = _SCALE * (t - _ALPHA)
        s_adj = s - jnp.exp(_SCALE * (t - m)) + jnp.exp(tgt - _SCALE * m)
        o_ref[...] = _SCALE * m + jnp.log(s_adj) - tgt


def kernel(score, y):
    batch, num_cls = score.shape
    y2 = y.reshape(-1, 1).astype(jnp.int32)
    ncol = pl.cdiv(num_cls, _CBLK)
    out = pl.pallas_call(
        _cos_loss_body,
        grid=(batch // _RBLK, ncol),
        in_specs=[
            pl.BlockSpec((_RBLK, 1), lambda i, j: (i, 0)),
            pl.BlockSpec((_RBLK, _CBLK), lambda i, j: (i, j)),
        ],
        out_specs=pl.BlockSpec((_RBLK, 1), lambda i, j: (i, 0)),
        out_shape=jax.ShapeDtypeStruct((batch, 1), score.dtype),
        scratch_shapes=[
            pltpu.VMEM((_RBLK, 1), jnp.float32),
            pltpu.VMEM((_RBLK, 1), jnp.float32),
            pltpu.VMEM((_RBLK, 1), jnp.float32),
        ],
        compiler_params=pltpu.CompilerParams(
            dimension_semantics=("parallel", "arbitrary"),
        ),
    )(y2, score)
    return out[:, 0]
